# Initial kernel scaffold; baseline (speedup 1.0000x reference)
import jax
import jax.numpy as jnp
from jax.experimental import pallas as pl


def kernel(x, emb_table, ltype_table, wtype_table, fc_w, fc_b, out_w, out_b):
    raise NotImplementedError("write your pallas kernel here")


# speedup vs baseline: 3.9165x; 3.9165x over previous
import functools

import jax
import jax.numpy as jnp
from jax import lax
from jax.experimental import pallas as pl
from jax.experimental.pallas import tpu as pltpu
from jax.experimental.pallas import tpu_sc as plsc

NUM_LAYERS = 42
NUM_LTYPES = 7
NUM_WTYPES = 7
HIDDEN = 16
BATCH = 16384

NC = 2
NS = 16
L = 16
NW = NC * NS
BPW = BATCH // NW
IDX_CHUNK = 128
NCHUNK = BPW // IDX_CHUNK


def _table_body(emb_ref, lt_ref, wt_ref, fcw_ref, fcb_ref, outw_ref, outb_ref,
                t_ref):
    outw = outw_ref[...]
    w1 = jnp.dot(fcw_ref[0:16, :], outw, preferred_element_type=jnp.float32)
    w2 = jnp.dot(fcw_ref[16:32, :], outw, preferred_element_type=jnp.float32)
    w3 = jnp.dot(fcw_ref[32:48, :], outw, preferred_element_type=jnp.float32)
    bias = (jnp.dot(fcb_ref[...], outw, preferred_element_type=jnp.float32)
            + outb_ref[...])
    a = jnp.dot(emb_ref[...], w1, preferred_element_type=jnp.float32) + bias
    b = jnp.dot(lt_ref[...], w2, preferred_element_type=jnp.float32)
    c = jnp.dot(wt_ref[...], w3, preferred_element_type=jnp.float32)
    t_ref[...] = (a[:, None, None, :] + b[None, :, None, :]
                  + c[None, None, :, :])


def _build_table(emb, lt, wt, fc_w, fc_b, out_w, out_b):
    t4 = pl.pallas_call(
        _table_body,
        out_shape=jax.ShapeDtypeStruct(
            (NUM_LAYERS, NUM_LTYPES, NUM_WTYPES, HIDDEN), jnp.float32),
    )(emb, lt, wt, fc_w, fc_b.reshape(1, HIDDEN), out_w,
      out_b.reshape(1, HIDDEN))
    return t4.reshape(NUM_LAYERS * NUM_LTYPES * NUM_WTYPES, HIDDEN)


def _sc_gather_body(t_hbm, xa_hbm, xb_hbm, xc_hbm, out_hbm,
                    xa_v, xb_v, xc_v, idx_v, rows_v, sem):
    wid = lax.axis_index("s") * NC + lax.axis_index("c")
    base = wid * BPW
    pltpu.sync_copy(xa_hbm.at[pl.ds(base, BPW)], xa_v)
    pltpu.sync_copy(xb_hbm.at[pl.ds(base, BPW)], xb_v)
    pltpu.sync_copy(xc_hbm.at[pl.ds(base, BPW)], xc_v)
    for i in range(BPW // L):
        s = pl.ds(i * L, L)
        row = i // (IDX_CHUNK // L)
        col = pl.ds((i % (IDX_CHUNK // L)) * L, L)
        idx_v[row, col] = (xa_v[s] * (NUM_LTYPES * NUM_WTYPES)
                           + xb_v[s] * NUM_WTYPES + xc_v[s])
    copies = [
        pltpu.async_copy(t_hbm.at[idx_v.at[j]],
                         rows_v.at[pl.ds(j * IDX_CHUNK, IDX_CHUNK)], sem)
        for j in range(NCHUNK)
    ]
    for cp in copies:
        cp.wait()
    pltpu.sync_copy(rows_v, out_hbm.at[pl.ds(base, BPW)])


@functools.lru_cache(maxsize=1)
def _make_sc_gather():
    mesh = plsc.VectorSubcoreMesh(
        core_axis_name="c", subcore_axis_name="s",
        num_cores=NC, num_subcores=NS)
    return pl.kernel(
        _sc_gather_body,
        out_type=jax.ShapeDtypeStruct((BATCH, HIDDEN), jnp.float32),
        mesh=mesh,
        scratch_types=[
            pltpu.VMEM((BPW,), jnp.int32),
            pltpu.VMEM((BPW,), jnp.int32),
            pltpu.VMEM((BPW,), jnp.int32),
            pltpu.VMEM((NCHUNK, IDX_CHUNK), jnp.int32),
            pltpu.VMEM((BPW, HIDDEN), jnp.float32),
            pltpu.SemaphoreType.DMA,
        ],
        compiler_params=pltpu.CompilerParams(use_tc_tiling_on_sc=False),
    )


def kernel(x, emb_table, ltype_table, wtype_table, fc_w, fc_b, out_w, out_b):
    x = x.astype(jnp.int32)
    t = _build_table(emb_table, ltype_table, wtype_table, fc_w, fc_b,
                     out_w, out_b)
    return _make_sc_gather()(t, x[:, 0], x[:, 1], x[:, 2])


# speedup vs baseline: 4.5223x; 1.1547x over previous
import functools

import jax
import jax.numpy as jnp
from jax import lax
from jax.experimental import pallas as pl
from jax.experimental.pallas import tpu as pltpu
from jax.experimental.pallas import tpu_sc as plsc

NUM_LAYERS = 42
NUM_LTYPES = 7
NUM_WTYPES = 7
HIDDEN = 16
BATCH = 16384

TROWS = NUM_LAYERS * NUM_LTYPES * NUM_WTYPES
TFLAT = TROWS * HIDDEN

NC = 2
NS = 16
L = 16
NW = NC * NS
BPW = BATCH // NW
NGROUP = BPW // L


def _table_body(emb_ref, lt_ref, wt_ref, fcw_ref, fcb_ref, outw_ref, outb_ref,
                t_ref):
    outw = outw_ref[...]
    w1 = jnp.dot(fcw_ref[0:16, :], outw, preferred_element_type=jnp.float32)
    w2 = jnp.dot(fcw_ref[16:32, :], outw, preferred_element_type=jnp.float32)
    w3 = jnp.dot(fcw_ref[32:48, :], outw, preferred_element_type=jnp.float32)
    bias = (jnp.dot(fcb_ref[...], outw, preferred_element_type=jnp.float32)
            + outb_ref[...])
    a = jnp.dot(emb_ref[...], w1, preferred_element_type=jnp.float32) + bias
    b = jnp.dot(lt_ref[...], w2, preferred_element_type=jnp.float32)
    c = jnp.dot(wt_ref[...], w3, preferred_element_type=jnp.float32)
    t_ref[...] = (a[:, None, None, :] + b[None, :, None, :]
                  + c[None, None, :, :])


def _build_table(emb, lt, wt, fc_w, fc_b, out_w, out_b):
    t4 = pl.pallas_call(
        _table_body,
        out_shape=jax.ShapeDtypeStruct(
            (NUM_LAYERS, NUM_LTYPES, NUM_WTYPES, HIDDEN), jnp.float32),
    )(emb, lt, wt, fc_w, fc_b.reshape(1, HIDDEN), out_w,
      out_b.reshape(1, HIDDEN))
    return t4.reshape(TFLAT)


def _sc_gather_body(t_hbm, xa_hbm, xb_hbm, xc_hbm, out_hbm,
                    tab_v, xa_v, xb_v, xc_v, outt_v):
    wid = lax.axis_index("s") * NC + lax.axis_index("c")
    base = wid * BPW
    pltpu.sync_copy(t_hbm, tab_v)
    pltpu.sync_copy(xa_hbm.at[pl.ds(base, BPW)], xa_v)
    pltpu.sync_copy(xb_hbm.at[pl.ds(base, BPW)], xb_v)
    pltpu.sync_copy(xc_hbm.at[pl.ds(base, BPW)], xc_v)
    for g in range(NGROUP):
        s = pl.ds(g * L, L)
        flat = (xa_v[s] * (NUM_LTYPES * NUM_WTYPES * HIDDEN)
                + xb_v[s] * (NUM_WTYPES * HIDDEN) + xc_v[s] * HIDDEN)
        for f in range(HIDDEN):
            outt_v[f, s] = plsc.load_gather(tab_v, [flat + f])
    pltpu.sync_copy(outt_v, out_hbm.at[:, pl.ds(base, BPW)])


@functools.lru_cache(maxsize=1)
def _make_sc_gather():
    mesh = plsc.VectorSubcoreMesh(
        core_axis_name="c", subcore_axis_name="s",
        num_cores=NC, num_subcores=NS)
    return pl.kernel(
        _sc_gather_body,
        out_type=jax.ShapeDtypeStruct((HIDDEN, BATCH), jnp.float32),
        mesh=mesh,
        scratch_types=[
            pltpu.VMEM((TFLAT,), jnp.float32),
            pltpu.VMEM((BPW,), jnp.int32),
            pltpu.VMEM((BPW,), jnp.int32),
            pltpu.VMEM((BPW,), jnp.int32),
            pltpu.VMEM((HIDDEN, BPW), jnp.float32),
        ],
        compiler_params=pltpu.CompilerParams(needs_layout_passes=False),
    )


def kernel(x, emb_table, ltype_table, wtype_table, fc_w, fc_b, out_w, out_b):
    x = x.astype(jnp.int32)
    t = _build_table(emb_table, ltype_table, wtype_table, fc_w, fc_b,
                     out_w, out_b)
    outt = _make_sc_gather()(t, x[:, 0], x[:, 1], x[:, 2])
    return outt.T
